# baseline (speedup 1.0000x reference)
import functools
import math

import jax
import jax.numpy as jnp
from jax import lax
from jax.experimental import pallas as pl
from jax.experimental.pallas import tpu as pltpu
from jax.experimental.pallas import tpu_sc as plsc

D = 64
SCALE = math.sqrt(D)
NC, NS, L = 2, 16, 16
NW = NC * NS
B = 4096 * 200
BPW = B // NW
C = 512
G = BPW // C


def _scale_rows(rows):
    @pl.loop(0, C, step=2)
    def _(r):
        for rr in range(2):
            for c16 in range(D // L):
                slc = (pl.ds(r + rr, 1), pl.ds(c16 * L, L))
                rows.at[slc][...] = rows.at[slc][...] * SCALE


def _embed_sc(table, idx):
    mesh = plsc.VectorSubcoreMesh(
        core_axis_name="c", subcore_axis_name="s", num_cores=NC, num_subcores=NS
    )

    @functools.partial(
        pl.kernel,
        out_type=jax.ShapeDtypeStruct((B, D), jnp.float32),
        mesh=mesh,
        compiler_params=pltpu.CompilerParams(use_tc_tiling_on_sc=False),
        scratch_types=[
            pltpu.VMEM((BPW,), jnp.int32),
            pltpu.VMEM((C, D), jnp.float32),
            pltpu.VMEM((C, D), jnp.float32),
            pltpu.SemaphoreType.DMA,
            pltpu.SemaphoreType.DMA,
            pltpu.SemaphoreType.DMA,
            pltpu.SemaphoreType.DMA,
        ],
    )
    def k(tab_hbm, idx_hbm, out_hbm, idx_v, rows0, rows1, g0, g1, s0, s1):
        wid = lax.axis_index("s") * NC + lax.axis_index("c")
        base = wid * BPW

        def start_gather(c, rows, sem):
            pltpu.async_copy(tab_hbm.at[idx_v.at[pl.ds(c * C, C)]], rows, sem)

        def wait_gather(rows, sem):
            pltpu.make_async_copy(tab_hbm.at[pl.ds(0, C)], rows, sem).wait()

        def start_store(c, rows, sem):
            pltpu.async_copy(rows, out_hbm.at[pl.ds(base + c * C, C)], sem)

        def wait_store(rows, sem):
            pltpu.make_async_copy(rows, out_hbm.at[pl.ds(base, C)], sem).wait()

        pltpu.sync_copy(idx_hbm.at[pl.ds(base, BPW)], idx_v)
        start_gather(0, rows0, g0)

        @pl.loop(0, G, step=2)
        def _(g):
            @pl.when(g > 0)
            def _():
                wait_store(rows1, s1)

            start_gather(g + 1, rows1, g1)
            wait_gather(rows0, g0)
            _scale_rows(rows0)
            start_store(g, rows0, s0)

            wait_gather(rows1, g1)
            _scale_rows(rows1)
            wait_store(rows0, s0)

            @pl.when(g + 2 < G)
            def _():
                start_gather(g + 2, rows0, g0)

            start_store(g + 1, rows1, s1)

        wait_store(rows1, s1)

    return k(table, idx)


def kernel(tokens, embedding):
    idx = tokens.reshape(-1).astype(jnp.int32)
    out = _embed_sc(embedding, idx)
    return out.reshape(*tokens.shape, D)
